# Initial kernel scaffold; baseline (speedup 1.0000x reference)
import jax
import jax.numpy as jnp
from jax.experimental import pallas as pl


def kernel(x, yolo_W1, yolo_b1, yolo_W2, yolo_b2, gnn_W1, gnn_b1, gnn_W2, gnn_b2, final_W, final_b):
    raise NotImplementedError("write your pallas kernel here")


# speedup vs baseline: 3.6346x; 3.6346x over previous
import functools

import jax
import jax.numpy as jnp
import numpy as np
from jax import lax
from jax.experimental import pallas as pl
from jax.experimental.pallas import tpu as pltpu
from jax.experimental.pallas import tpu_sc as plsc

F32 = jnp.float32

_hdot = functools.partial(jnp.dot, precision=lax.Precision.HIGHEST,
                          preferred_element_type=F32)


def _pool_body(x_ref, lmat_ref, pmat_ref, o_ref):
    xb = x_ref[0]
    t = _hdot(lmat_ref[...], xb)
    o_ref[0] = _hdot(t, pmat_ref[...]) * (1.0 / 196.0)


def _pool(x2, lmat, pmat):
    bn = x2.shape[0]
    return pl.pallas_call(
        _pool_body,
        grid=(bn,),
        in_specs=[
            pl.BlockSpec((1, 672, 224), lambda b: (b, 0, 0)),
            pl.BlockSpec((48, 672), lambda b: (0, 0)),
            pl.BlockSpec((224, 16), lambda b: (0, 0)),
        ],
        out_specs=pl.BlockSpec((1, 48, 16), lambda b: (b, 0, 0)),
        out_shape=jax.ShapeDtypeStruct((bn, 48, 16), F32),
    )(x2, lmat, pmat)


def _route_body(p_ref, w1_ref, b1_ref, w2_ref, b2_ref,
                xg_ref, top2_ref, m_ref, idx_ref):
    pv = p_ref[0]
    f = jax.nn.relu(_hdot(pv, w1_ref[...]) + b1_ref[...])
    lg = _hdot(f, w2_ref[...]) + b2_ref[...]

    io64 = lax.broadcasted_iota(jnp.int32, (1, 64), 1)
    m1 = jnp.max(lg, axis=1, keepdims=True)
    i1 = jnp.min(jnp.where(lg == m1, io64, 64), axis=1, keepdims=True)
    lg2 = jnp.where(io64 == i1, F32(-1e30), lg)
    m2 = jnp.max(lg2, axis=1, keepdims=True)
    i2 = jnp.min(jnp.where(lg2 == m2, io64, 64), axis=1, keepdims=True)
    io128 = lax.broadcasted_iota(jnp.int32, (1, 128), 1)
    top2_ref[0] = jnp.where(io128 == 0,
                            jnp.broadcast_to(i1, (1, 128)),
                            jnp.broadcast_to(i2, (1, 128)))
    io64r = lax.broadcasted_iota(jnp.int32, (1, 64), 1)
    e_sel = jnp.where(io64r < 32,
                      jnp.broadcast_to(i1, (1, 64)),
                      jnp.broadcast_to(i2, (1, 64)))
    idx_ref[0] = e_sel * 32 + (io64r & 31)

    parts = [f]
    for k in range(4):
        q = f[:, k * 128:(k + 1) * 128]
        parts.append(jnp.pad(q, ((0, 0), (0, 384))))
    xg = jnp.concatenate(parts, axis=0)
    xg_ref[0] = xg

    diff = xg[:, None, :] - xg[None, :, :]
    d2 = jnp.sum(diff * diff, axis=-1)
    io5c = lax.broadcasted_iota(jnp.int32, (5, 5), 1)
    io5r = lax.broadcasted_iota(jnp.int32, (5, 5), 0)
    mx = jnp.max(d2, axis=1, keepdims=True)
    excl = jnp.max(jnp.where(d2 == mx, io5c, -1), axis=1, keepdims=True)
    m_ref[0] = ((io5c != excl).astype(F32) + (io5c == io5r).astype(F32))


def _route(p, w1, b1, w2, b2):
    bn = p.shape[0]
    return pl.pallas_call(
        _route_body,
        grid=(bn,),
        in_specs=[
            pl.BlockSpec((1, 1, 768), lambda b: (b, 0, 0)),
            pl.BlockSpec((768, 512), lambda b: (0, 0)),
            pl.BlockSpec((1, 512), lambda b: (0, 0)),
            pl.BlockSpec((512, 64), lambda b: (0, 0)),
            pl.BlockSpec((1, 64), lambda b: (0, 0)),
        ],
        out_specs=[
            pl.BlockSpec((1, 5, 512), lambda b: (b, 0, 0)),
            pl.BlockSpec((1, 1, 128), lambda b: (b, 0, 0)),
            pl.BlockSpec((1, 5, 5), lambda b: (b, 0, 0)),
            pl.BlockSpec((1, 1, 64), lambda b: (b, 0, 0)),
        ],
        out_shape=[
            jax.ShapeDtypeStruct((bn, 5, 512), F32),
            jax.ShapeDtypeStruct((bn, 1, 128), jnp.int32),
            jax.ShapeDtypeStruct((bn, 5, 5), F32),
            jax.ShapeDtypeStruct((bn, 1, 64), jnp.int32),
        ],
    )(p.reshape(bn, 1, 768), w1, b1.reshape(1, 512), w2, b2.reshape(1, 64))


def _sc_gather(idx_flat, top2_flat, w1_view, w2_view, b1, b2):
    mesh = plsc.VectorSubcoreMesh(core_axis_name="c", subcore_axis_name="s")

    @functools.partial(
        pl.kernel,
        out_type=(
            jax.ShapeDtypeStruct((512, 4096), F32),
            jax.ShapeDtypeStruct((512, 1024), F32),
            jax.ShapeDtypeStruct((16, 256), F32),
            jax.ShapeDtypeStruct((16, 128), F32),
        ),
        mesh=mesh,
        scratch_types=(
            pltpu.VMEM((16,), jnp.int32),
            pltpu.VMEM((16,), jnp.int32),
            pltpu.VMEM((16, 4096), F32),
            pltpu.VMEM((16, 1024), F32),
            pltpu.VMEM((16, 256), F32),
            pltpu.VMEM((16, 128), F32),
            pltpu.SemaphoreType.DMA,
            pltpu.SemaphoreType.DMA,
        ),
    )
    def gather_k(idx_hbm, top2_hbm, w1_hbm, w2_hbm, b1_hbm, b2_hbm,
                 o_w1, o_w2, o_b1, o_b2,
                 e_v, idx_v, rows1_v, rows2_v, b1_v, b2_v, sem, semb):
        wid = lax.axis_index("s") * 2 + lax.axis_index("c")
        pltpu.sync_copy(idx_hbm.at[pl.ds(wid * 16, 16)], idx_v)
        pltpu.async_copy(w1_hbm.at[idx_v], rows1_v, sem).wait()
        pltpu.sync_copy(rows1_v, o_w1.at[pl.ds(wid * 16, 16)])
        pltpu.async_copy(w2_hbm.at[idx_v], rows2_v, sem).wait()
        pltpu.sync_copy(rows2_v, o_w2.at[pl.ds(wid * 16, 16)])

        @pl.when(wid == 0)
        def _():
            pltpu.sync_copy(top2_hbm, e_v)
            pltpu.async_copy(b1_hbm.at[e_v], b1_v, semb).wait()
            pltpu.sync_copy(b1_v, o_b1)

        @pl.when(wid == 1)
        def _():
            pltpu.sync_copy(top2_hbm, e_v)
            pltpu.async_copy(b2_hbm.at[e_v], b2_v, semb).wait()
            pltpu.sync_copy(b2_v, o_b2)

    return gather_k(idx_flat, top2_flat, w1_view, w2_view, b1, b2)


def _experts_body(xg_ref, m_ref, w1_ref, w2_ref, b1_ref, b2_ref,
                  fw_ref, fb_ref, o_ref, acc_ref):
    t = pl.program_id(0)
    xg = xg_ref[0]
    mm = m_ref[0]
    xw = _hdot(xg, w1_ref[0])
    h = jax.nn.relu(_hdot(mm, xw) * 0.2 + b1_ref[0])
    h2 = _hdot(mm, _hdot(h, w2_ref[0])) * 0.2 + b2_ref[0]
    acc_ref[pl.ds(t, 1), :] = jnp.mean(h2, axis=0, keepdims=True)

    @pl.when(t == 15)
    def _():
        comb = acc_ref[...]
        fin = _hdot(comb, fw_ref[...])
        io5c = lax.broadcasted_iota(jnp.int32, (5, 5), 1)
        io5r = lax.broadcasted_iota(jnp.int32, (5, 5), 0)
        c7 = mm - (io5c == io5r).astype(F32)
        c7p = jnp.pad(c7, ((0, 11), (0, 11)))
        r16 = lax.broadcasted_iota(jnp.int32, (16, 16), 0)
        c16 = lax.broadcasted_iota(jnp.int32, (16, 16), 1)
        diag = jnp.where(r16 == c16,
                         jnp.where(r16 < 5, F32(0.2), F32(1.0)), F32(0.0))
        mf = diag + c7p * 0.2
        fin2 = _hdot(mf, fin) + fb_ref[...]
        r8 = lax.broadcasted_iota(jnp.int32, (8, 16), 0)
        c8 = lax.broadcasted_iota(jnp.int32, (8, 16), 1)
        pairmean = ((c8 == 2 * r8) | (c8 == 2 * r8 + 1)).astype(F32)
        o_ref[...] = _hdot(pairmean, fin2) * 0.5


def _experts(xg, m, w1g, w2g, b1g, b2g, fw, fb):
    return pl.pallas_call(
        _experts_body,
        grid=(16,),
        in_specs=[
            pl.BlockSpec((1, 5, 512), lambda t: (t // 2, 0, 0)),
            pl.BlockSpec((1, 5, 5), lambda t: (t // 2, 0, 0)),
            pl.BlockSpec((1, 512, 256), lambda t: (t, 0, 0)),
            pl.BlockSpec((1, 256, 128), lambda t: (t, 0, 0)),
            pl.BlockSpec((1, 1, 256), lambda t: (t, 0, 0)),
            pl.BlockSpec((1, 1, 128), lambda t: (t, 0, 0)),
            pl.BlockSpec((128, 64), lambda t: (0, 0)),
            pl.BlockSpec((1, 64), lambda t: (0, 0)),
        ],
        out_specs=pl.BlockSpec((8, 64), lambda t: (0, 0)),
        out_shape=jax.ShapeDtypeStruct((8, 64), F32),
        scratch_shapes=[pltpu.VMEM((16, 128), F32)],
    )(xg, m, w1g, w2g, b1g, b2g, fw, fb)


def _make_pool_consts():
    lmat = np.zeros((48, 672), dtype=np.float32)
    for a in range(48):
        ch, i = divmod(a, 16)
        lmat[a, ch * 224 + i * 14:(ch * 224 + (i + 1) * 14)] = 1.0
    pmat = np.zeros((224, 16), dtype=np.float32)
    for rr in range(224):
        pmat[rr, rr // 14] = 1.0
    return jnp.asarray(lmat), jnp.asarray(pmat)


def kernel(x, yolo_W1, yolo_b1, yolo_W2, yolo_b2,
           gnn_W1, gnn_b1, gnn_W2, gnn_b2, final_W, final_b):
    bn = x.shape[0]
    lmat, pmat = _make_pool_consts()
    p48 = _pool(x.reshape(bn, 672, 224), lmat, pmat)
    p = p48.reshape(bn, 768)
    xg, top2_3d, m, idx_3d = _route(p, yolo_W1, yolo_b1, yolo_W2, yolo_b2)
    top2_flat = top2_3d[:, 0, :2].reshape(2 * bn).astype(jnp.int32)
    idx_flat = idx_3d.reshape(64 * bn)

    o_w1, o_w2, b1g, b2g = _sc_gather(
        idx_flat, top2_flat,
        gnn_W1.reshape(64 * 32, 4096),
        gnn_W2.reshape(64 * 32, 1024),
        gnn_b1, gnn_b2)
    w1g = o_w1.reshape(16, 512, 256)
    w2g = o_w2.reshape(16, 256, 128)

    return _experts(xg, m, w1g, w2g,
                    b1g.reshape(16, 1, 256), b2g.reshape(16, 1, 128),
                    final_W, final_b.reshape(1, 64))


# speedup vs baseline: 6.1161x; 1.6827x over previous
import functools

import jax
import jax.numpy as jnp
import numpy as np
from jax import lax
from jax.experimental import pallas as pl
from jax.experimental.pallas import tpu as pltpu
from jax.experimental.pallas import tpu_sc as plsc

F32 = jnp.float32

_hdot = functools.partial(jnp.dot, precision=lax.Precision.HIGHEST,
                          preferred_element_type=F32)


def _route_body(x_ref, lmat_ref, pmat_ref, w1_ref, b1_ref, w2_ref, b2_ref,
                xg_ref, top2_ref, m_ref, idx1_ref, idx2_ref):
    xb = x_ref[0]
    z = _hdot(lmat_ref[...], xb)
    pooled = _hdot(z, pmat_ref[...]) * (1.0 / 196.0)

    acc = b1_ref[...]
    for a in range(48):
        acc = acc + _hdot(pooled[a:a + 1, :], w1_ref[a])
    f = jax.nn.relu(acc)
    lg = _hdot(f, w2_ref[...]) + b2_ref[...]

    io64 = lax.broadcasted_iota(jnp.int32, (1, 64), 1)
    m1 = jnp.max(lg, axis=1, keepdims=True)
    i1 = jnp.min(jnp.where(lg == m1, io64, 64), axis=1, keepdims=True)
    lg2 = jnp.where(io64 == i1, F32(-1e30), lg)
    m2 = jnp.max(lg2, axis=1, keepdims=True)
    i2 = jnp.min(jnp.where(lg2 == m2, io64, 64), axis=1, keepdims=True)
    io128 = lax.broadcasted_iota(jnp.int32, (1, 128), 1)
    top2_ref[0] = jnp.where(io128 == 0,
                            jnp.broadcast_to(i1, (1, 128)),
                            jnp.broadcast_to(i2, (1, 128)))

    ioa = lax.broadcasted_iota(jnp.int32, (1, 1024), 1)
    e_sel = jnp.where(ioa < 512,
                      jnp.broadcast_to(i1, (1, 1024)),
                      jnp.broadcast_to(i2, (1, 1024)))
    idx1_ref[0] = e_sel * 512 + (ioa & 511)
    iob = lax.broadcasted_iota(jnp.int32, (1, 512), 1)
    e_selb = jnp.where(iob < 256,
                       jnp.broadcast_to(i1, (1, 512)),
                       jnp.broadcast_to(i2, (1, 512)))
    idx2_ref[0] = e_selb * 256 + (iob & 255)

    parts = [f]
    for k in range(4):
        q = f[:, k * 128:(k + 1) * 128]
        parts.append(jnp.pad(q, ((0, 0), (0, 384))))
    xg = jnp.concatenate(parts, axis=0)
    xg_ref[0] = xg

    diff = xg[:, None, :] - xg[None, :, :]
    d2 = jnp.sum(diff * diff, axis=-1)
    io5c = lax.broadcasted_iota(jnp.int32, (5, 5), 1)
    io5r = lax.broadcasted_iota(jnp.int32, (5, 5), 0)
    mx = jnp.max(d2, axis=1, keepdims=True)
    excl = jnp.max(jnp.where(d2 == mx, io5c, -1), axis=1, keepdims=True)
    m_ref[0] = ((io5c != excl).astype(F32) + (io5c == io5r).astype(F32))


def _route(x3, lmat, pmat, w1, b1, w2, b2):
    bn = x3.shape[0]
    return pl.pallas_call(
        _route_body,
        grid=(bn,),
        in_specs=[
            pl.BlockSpec((1, 672, 224), lambda b: (b, 0, 0)),
            pl.BlockSpec((48, 672), lambda b: (0, 0)),
            pl.BlockSpec((224, 16), lambda b: (0, 0)),
            pl.BlockSpec((48, 16, 512), lambda b: (0, 0, 0)),
            pl.BlockSpec((1, 512), lambda b: (0, 0)),
            pl.BlockSpec((512, 64), lambda b: (0, 0)),
            pl.BlockSpec((1, 64), lambda b: (0, 0)),
        ],
        out_specs=[
            pl.BlockSpec((1, 5, 512), lambda b: (b, 0, 0)),
            pl.BlockSpec((1, 1, 128), lambda b: (b, 0, 0)),
            pl.BlockSpec((1, 5, 5), lambda b: (b, 0, 0)),
            pl.BlockSpec((1, 1, 1024), lambda b: (b, 0, 0)),
            pl.BlockSpec((1, 1, 512), lambda b: (b, 0, 0)),
        ],
        out_shape=[
            jax.ShapeDtypeStruct((bn, 5, 512), F32),
            jax.ShapeDtypeStruct((bn, 1, 128), jnp.int32),
            jax.ShapeDtypeStruct((bn, 5, 5), F32),
            jax.ShapeDtypeStruct((bn, 1, 1024), jnp.int32),
            jax.ShapeDtypeStruct((bn, 1, 512), jnp.int32),
        ],
    )(x3, lmat, pmat, w1.reshape(48, 16, 512),
      b1.reshape(1, 512), w2, b2.reshape(1, 64))


def _sc_gather(idx1_flat, idx2_flat, top2_flat, w1_view, w2_view, b1, b2):
    mesh = plsc.VectorSubcoreMesh(core_axis_name="c", subcore_axis_name="s")

    @functools.partial(
        pl.kernel,
        out_type=(
            jax.ShapeDtypeStruct((8192, 256), F32),
            jax.ShapeDtypeStruct((4096, 128), F32),
            jax.ShapeDtypeStruct((16, 256), F32),
            jax.ShapeDtypeStruct((16, 128), F32),
        ),
        mesh=mesh,
        scratch_types=(
            pltpu.VMEM((128,), jnp.int32),
            pltpu.VMEM((128,), jnp.int32),
            pltpu.VMEM((128,), jnp.int32),
            pltpu.VMEM((16,), jnp.int32),
            pltpu.VMEM((256, 256), F32),
            pltpu.VMEM((128, 128), F32),
            pltpu.VMEM((16, 256), F32),
            pltpu.VMEM((16, 128), F32),
            pltpu.SemaphoreType.DMA,
            pltpu.SemaphoreType.DMA,
        ),
    )
    def gather_k(idx1_hbm, idx2_hbm, top2_hbm, w1_hbm, w2_hbm, b1_hbm, b2_hbm,
                 o_w1, o_w2, o_b1, o_b2,
                 ia_v, ib_v, ic_v, e_v, rows1_v, rows2_v, b1_v, b2_v,
                 sem, semb):
        wid = lax.axis_index("s") * 2 + lax.axis_index("c")
        pltpu.sync_copy(idx1_hbm.at[pl.ds(wid * 256, 128)], ia_v)
        pltpu.sync_copy(idx1_hbm.at[pl.ds(wid * 256 + 128, 128)], ib_v)
        pltpu.sync_copy(idx2_hbm.at[pl.ds(wid * 128, 128)], ic_v)
        c1 = pltpu.async_copy(w1_hbm.at[ia_v], rows1_v.at[pl.ds(0, 128)], sem)
        c2 = pltpu.async_copy(w1_hbm.at[ib_v], rows1_v.at[pl.ds(128, 128)], sem)
        c3 = pltpu.async_copy(w2_hbm.at[ic_v], rows2_v, sem)
        c1.wait()
        c2.wait()
        c3.wait()
        pltpu.sync_copy(rows1_v, o_w1.at[pl.ds(wid * 256, 256)])
        pltpu.sync_copy(rows2_v, o_w2.at[pl.ds(wid * 128, 128)])

        @pl.when(wid == 0)
        def _():
            pltpu.sync_copy(top2_hbm, e_v)
            pltpu.async_copy(b1_hbm.at[e_v], b1_v, semb).wait()
            pltpu.sync_copy(b1_v, o_b1)

        @pl.when(wid == 1)
        def _():
            pltpu.sync_copy(top2_hbm, e_v)
            pltpu.async_copy(b2_hbm.at[e_v], b2_v, semb).wait()
            pltpu.sync_copy(b2_v, o_b2)

    return gather_k(idx1_flat, idx2_flat, top2_flat, w1_view, w2_view, b1, b2)


def _experts_body(xg_ref, m_ref, w1_ref, w2_ref, b1_ref, b2_ref,
                  fw_ref, fb_ref, o_ref, acc_ref):
    t = pl.program_id(0)
    xg = xg_ref[0]
    mm = m_ref[0]
    xw = _hdot(xg, w1_ref[0])
    h = jax.nn.relu(_hdot(mm, xw) * 0.2 + b1_ref[pl.ds(t, 1), :])
    h2 = _hdot(mm, _hdot(h, w2_ref[0])) * 0.2 + b2_ref[pl.ds(t, 1), :]
    acc_ref[pl.ds(t, 1), :] = jnp.mean(h2, axis=0, keepdims=True)

    @pl.when(t == 15)
    def _():
        comb = acc_ref[...]
        fin = _hdot(comb, fw_ref[...])
        io5c = lax.broadcasted_iota(jnp.int32, (5, 5), 1)
        io5r = lax.broadcasted_iota(jnp.int32, (5, 5), 0)
        c7 = mm - (io5c == io5r).astype(F32)
        c7p = jnp.pad(c7, ((0, 11), (0, 11)))
        r16 = lax.broadcasted_iota(jnp.int32, (16, 16), 0)
        c16 = lax.broadcasted_iota(jnp.int32, (16, 16), 1)
        diag = jnp.where(r16 == c16,
                         jnp.where(r16 < 5, F32(0.2), F32(1.0)), F32(0.0))
        mf = diag + c7p * 0.2
        fin2 = _hdot(mf, fin) + fb_ref[...]
        r8 = lax.broadcasted_iota(jnp.int32, (8, 16), 0)
        c8 = lax.broadcasted_iota(jnp.int32, (8, 16), 1)
        pairmean = ((c8 == 2 * r8) | (c8 == 2 * r8 + 1)).astype(F32)
        o_ref[...] = _hdot(pairmean, fin2) * 0.5


def _experts(xg, m, w1g, w2g, b1g, b2g, fw, fb):
    return pl.pallas_call(
        _experts_body,
        grid=(16,),
        in_specs=[
            pl.BlockSpec((1, 5, 512), lambda t: (t // 2, 0, 0)),
            pl.BlockSpec((1, 5, 5), lambda t: (t // 2, 0, 0)),
            pl.BlockSpec((1, 512, 256), lambda t: (t, 0, 0)),
            pl.BlockSpec((1, 256, 128), lambda t: (t, 0, 0)),
            pl.BlockSpec((16, 256), lambda t: (0, 0)),
            pl.BlockSpec((16, 128), lambda t: (0, 0)),
            pl.BlockSpec((128, 64), lambda t: (0, 0)),
            pl.BlockSpec((1, 64), lambda t: (0, 0)),
        ],
        out_specs=pl.BlockSpec((8, 64), lambda t: (0, 0)),
        out_shape=jax.ShapeDtypeStruct((8, 64), F32),
        scratch_shapes=[pltpu.VMEM((16, 128), F32)],
    )(xg, m, w1g, w2g, b1g, b2g, fw, fb)


def _make_pool_consts():
    lmat = np.zeros((48, 672), dtype=np.float32)
    for a in range(48):
        ch, i = divmod(a, 16)
        lmat[a, ch * 224 + i * 14:(ch * 224 + (i + 1) * 14)] = 1.0
    pmat = np.zeros((224, 16), dtype=np.float32)
    for rr in range(224):
        pmat[rr, rr // 14] = 1.0
    return jnp.asarray(lmat), jnp.asarray(pmat)


def kernel(x, yolo_W1, yolo_b1, yolo_W2, yolo_b2,
           gnn_W1, gnn_b1, gnn_W2, gnn_b2, final_W, final_b):
    bn = x.shape[0]
    lmat, pmat = _make_pool_consts()
    xg, top2_3d, m, idx1_3d, idx2_3d = _route(
        x.reshape(bn, 672, 224), lmat, pmat,
        yolo_W1, yolo_b1, yolo_W2, yolo_b2)
    top2_flat = top2_3d[:, 0, :2].reshape(2 * bn).astype(jnp.int32)

    o_w1, o_w2, b1g, b2g = _sc_gather(
        idx1_3d.reshape(1024 * bn), idx2_3d.reshape(512 * bn), top2_flat,
        gnn_W1.reshape(64 * 512, 256),
        gnn_W2.reshape(64 * 256, 128),
        gnn_b1, gnn_b2)
    w1g = o_w1.reshape(16, 512, 256)
    w2g = o_w2.reshape(16, 256, 128)

    return _experts(xg, m, w1g, w2g, b1g, b2g,
                    final_W, final_b.reshape(1, 64))


# speedup vs baseline: 6.3981x; 1.0461x over previous
import functools

import jax
import jax.numpy as jnp
import numpy as np
from jax import lax
from jax.experimental import pallas as pl
from jax.experimental.pallas import tpu as pltpu
from jax.experimental.pallas import tpu_sc as plsc

F32 = jnp.float32

_hdot = functools.partial(jnp.dot, precision=lax.Precision.HIGHEST,
                          preferred_element_type=F32)
_fdot = functools.partial(jnp.dot, precision=lax.Precision.DEFAULT,
                          preferred_element_type=F32)


def _route_body(x_ref, lmat_ref, pmat_ref, w1_ref, b1_ref, w2_ref, b2_ref,
                xg_ref, top2_ref, m_ref, idx1_ref, idx2_ref):
    xb = x_ref[0]
    z = _hdot(lmat_ref[...], xb)
    pooled = _hdot(z, pmat_ref[...]) * (1.0 / 196.0)

    acc = b1_ref[...]
    for a in range(48):
        acc = acc + _hdot(pooled[a:a + 1, :], w1_ref[a])
    f = jax.nn.relu(acc)
    lg = _hdot(f, w2_ref[...]) + b2_ref[...]

    io64 = lax.broadcasted_iota(jnp.int32, (1, 64), 1)
    m1 = jnp.max(lg, axis=1, keepdims=True)
    i1 = jnp.min(jnp.where(lg == m1, io64, 64), axis=1, keepdims=True)
    lg2 = jnp.where(io64 == i1, F32(-1e30), lg)
    m2 = jnp.max(lg2, axis=1, keepdims=True)
    i2 = jnp.min(jnp.where(lg2 == m2, io64, 64), axis=1, keepdims=True)
    io128 = lax.broadcasted_iota(jnp.int32, (1, 128), 1)
    top2_ref[0] = jnp.where(io128 == 0,
                            jnp.broadcast_to(i1, (1, 128)),
                            jnp.broadcast_to(i2, (1, 128)))

    ioa = lax.broadcasted_iota(jnp.int32, (1, 1024), 1)
    e_sel = jnp.where(ioa < 512,
                      jnp.broadcast_to(i1, (1, 1024)),
                      jnp.broadcast_to(i2, (1, 1024)))
    idx1_ref[0] = e_sel * 512 + (ioa & 511)
    iob = lax.broadcasted_iota(jnp.int32, (1, 512), 1)
    e_selb = jnp.where(iob < 256,
                       jnp.broadcast_to(i1, (1, 512)),
                       jnp.broadcast_to(i2, (1, 512)))
    idx2_ref[0] = e_selb * 256 + (iob & 255)

    parts = [f]
    for k in range(4):
        q = f[:, k * 128:(k + 1) * 128]
        parts.append(jnp.pad(q, ((0, 0), (0, 384))))
    xg = jnp.concatenate(parts, axis=0)
    xg_ref[0] = xg

    diff = xg[:, None, :] - xg[None, :, :]
    d2 = jnp.sum(diff * diff, axis=-1)
    io5c = lax.broadcasted_iota(jnp.int32, (5, 5), 1)
    io5r = lax.broadcasted_iota(jnp.int32, (5, 5), 0)
    mx = jnp.max(d2, axis=1, keepdims=True)
    excl = jnp.max(jnp.where(d2 == mx, io5c, -1), axis=1, keepdims=True)
    m_ref[0] = ((io5c != excl).astype(F32) + (io5c == io5r).astype(F32))


def _route(x3, lmat, pmat, w1, b1, w2, b2):
    bn = x3.shape[0]
    return pl.pallas_call(
        _route_body,
        grid=(bn,),
        in_specs=[
            pl.BlockSpec((1, 672, 224), lambda b: (b, 0, 0)),
            pl.BlockSpec((48, 672), lambda b: (0, 0)),
            pl.BlockSpec((224, 16), lambda b: (0, 0)),
            pl.BlockSpec((48, 16, 512), lambda b: (0, 0, 0)),
            pl.BlockSpec((1, 512), lambda b: (0, 0)),
            pl.BlockSpec((512, 64), lambda b: (0, 0)),
            pl.BlockSpec((1, 64), lambda b: (0, 0)),
        ],
        out_specs=[
            pl.BlockSpec((1, 5, 512), lambda b: (b, 0, 0)),
            pl.BlockSpec((1, 1, 128), lambda b: (b, 0, 0)),
            pl.BlockSpec((1, 5, 5), lambda b: (b, 0, 0)),
            pl.BlockSpec((1, 1, 1024), lambda b: (b, 0, 0)),
            pl.BlockSpec((1, 1, 512), lambda b: (b, 0, 0)),
        ],
        out_shape=[
            jax.ShapeDtypeStruct((bn, 5, 512), F32),
            jax.ShapeDtypeStruct((bn, 1, 128), jnp.int32),
            jax.ShapeDtypeStruct((bn, 5, 5), F32),
            jax.ShapeDtypeStruct((bn, 1, 1024), jnp.int32),
            jax.ShapeDtypeStruct((bn, 1, 512), jnp.int32),
        ],
    )(x3, lmat, pmat, w1.reshape(48, 16, 512),
      b1.reshape(1, 512), w2, b2.reshape(1, 64))


def _sc_gather(idx1_flat, idx2_flat, top2_flat, w1_view, w2_view, b1, b2):
    mesh = plsc.VectorSubcoreMesh(core_axis_name="c", subcore_axis_name="s")

    @functools.partial(
        pl.kernel,
        out_type=(
            jax.ShapeDtypeStruct((8192, 256), F32),
            jax.ShapeDtypeStruct((4096, 128), F32),
            jax.ShapeDtypeStruct((16, 256), F32),
            jax.ShapeDtypeStruct((16, 128), F32),
        ),
        mesh=mesh,
        scratch_types=(
            pltpu.VMEM((128,), jnp.int32),
            pltpu.VMEM((128,), jnp.int32),
            pltpu.VMEM((128,), jnp.int32),
            pltpu.VMEM((16,), jnp.int32),
            pltpu.VMEM((256, 256), F32),
            pltpu.VMEM((128, 128), F32),
            pltpu.VMEM((16, 256), F32),
            pltpu.VMEM((16, 128), F32),
            pltpu.SemaphoreType.DMA,
            pltpu.SemaphoreType.DMA,
        ),
    )
    def gather_k(idx1_hbm, idx2_hbm, top2_hbm, w1_hbm, w2_hbm, b1_hbm, b2_hbm,
                 o_w1, o_w2, o_b1, o_b2,
                 ia_v, ib_v, ic_v, e_v, rows1_v, rows2_v, b1_v, b2_v,
                 sem, semb):
        wid = lax.axis_index("s") * 2 + lax.axis_index("c")
        pltpu.sync_copy(idx1_hbm.at[pl.ds(wid * 256, 128)], ia_v)
        pltpu.sync_copy(idx1_hbm.at[pl.ds(wid * 256 + 128, 128)], ib_v)
        pltpu.sync_copy(idx2_hbm.at[pl.ds(wid * 128, 128)], ic_v)
        c1 = pltpu.async_copy(w1_hbm.at[ia_v], rows1_v.at[pl.ds(0, 128)], sem)
        c2 = pltpu.async_copy(w1_hbm.at[ib_v], rows1_v.at[pl.ds(128, 128)], sem)
        c3 = pltpu.async_copy(w2_hbm.at[ic_v], rows2_v, sem)
        c1.wait()
        c2.wait()
        c3.wait()
        pltpu.sync_copy(rows1_v, o_w1.at[pl.ds(wid * 256, 256)])
        pltpu.sync_copy(rows2_v, o_w2.at[pl.ds(wid * 128, 128)])

        @pl.when(wid == 0)
        def _():
            pltpu.sync_copy(top2_hbm, e_v)
            pltpu.async_copy(b1_hbm.at[e_v], b1_v, semb).wait()
            pltpu.sync_copy(b1_v, o_b1)

        @pl.when(wid == 1)
        def _():
            pltpu.sync_copy(top2_hbm, e_v)
            pltpu.async_copy(b2_hbm.at[e_v], b2_v, semb).wait()
            pltpu.sync_copy(b2_v, o_b2)

    return gather_k(idx1_flat, idx2_flat, top2_flat, w1_view, w2_view, b1, b2)


def _experts_body(xg_ref, m_ref, w1_ref, w2_ref, b1_ref, b2_ref,
                  fw_ref, fb_ref, o_ref, acc_ref):
    t = pl.program_id(0)
    xg = xg_ref[0]
    mm = m_ref[0]
    xw = _fdot(xg, w1_ref[0])
    h = jax.nn.relu(_fdot(mm, xw) * 0.2 + b1_ref[pl.ds(t, 1), :])
    h2 = _fdot(mm, _fdot(h, w2_ref[0])) * 0.2 + b2_ref[pl.ds(t, 1), :]
    acc_ref[pl.ds(t, 1), :] = jnp.mean(h2, axis=0, keepdims=True)

    @pl.when(t == 15)
    def _():
        comb = acc_ref[...]
        fin = _fdot(comb, fw_ref[...])
        io5c = lax.broadcasted_iota(jnp.int32, (5, 5), 1)
        io5r = lax.broadcasted_iota(jnp.int32, (5, 5), 0)
        c7 = mm - (io5c == io5r).astype(F32)
        c7p = jnp.pad(c7, ((0, 11), (0, 11)))
        r16 = lax.broadcasted_iota(jnp.int32, (16, 16), 0)
        c16 = lax.broadcasted_iota(jnp.int32, (16, 16), 1)
        diag = jnp.where(r16 == c16,
                         jnp.where(r16 < 5, F32(0.2), F32(1.0)), F32(0.0))
        mf = diag + c7p * 0.2
        fin2 = _fdot(mf, fin) + fb_ref[...]
        r8 = lax.broadcasted_iota(jnp.int32, (8, 16), 0)
        c8 = lax.broadcasted_iota(jnp.int32, (8, 16), 1)
        pairmean = ((c8 == 2 * r8) | (c8 == 2 * r8 + 1)).astype(F32)
        o_ref[...] = _fdot(pairmean, fin2) * 0.5


def _experts(xg, m, w1g, w2g, b1g, b2g, fw, fb):
    return pl.pallas_call(
        _experts_body,
        grid=(16,),
        in_specs=[
            pl.BlockSpec((1, 5, 512), lambda t: (t // 2, 0, 0)),
            pl.BlockSpec((1, 5, 5), lambda t: (t // 2, 0, 0)),
            pl.BlockSpec((1, 512, 256), lambda t: (t, 0, 0)),
            pl.BlockSpec((1, 256, 128), lambda t: (t, 0, 0)),
            pl.BlockSpec((16, 256), lambda t: (0, 0)),
            pl.BlockSpec((16, 128), lambda t: (0, 0)),
            pl.BlockSpec((128, 64), lambda t: (0, 0)),
            pl.BlockSpec((1, 64), lambda t: (0, 0)),
        ],
        out_specs=pl.BlockSpec((8, 64), lambda t: (0, 0)),
        out_shape=jax.ShapeDtypeStruct((8, 64), F32),
        scratch_shapes=[pltpu.VMEM((16, 128), F32)],
    )(xg, m, w1g, w2g, b1g, b2g, fw, fb)


def _make_pool_consts():
    lmat = np.zeros((48, 672), dtype=np.float32)
    for a in range(48):
        ch, i = divmod(a, 16)
        lmat[a, ch * 224 + i * 14:(ch * 224 + (i + 1) * 14)] = 1.0
    pmat = np.zeros((224, 16), dtype=np.float32)
    for rr in range(224):
        pmat[rr, rr // 14] = 1.0
    return jnp.asarray(lmat), jnp.asarray(pmat)


def kernel(x, yolo_W1, yolo_b1, yolo_W2, yolo_b2,
           gnn_W1, gnn_b1, gnn_W2, gnn_b2, final_W, final_b):
    bn = x.shape[0]
    lmat, pmat = _make_pool_consts()
    xg, top2_3d, m, idx1_3d, idx2_3d = _route(
        x.reshape(bn, 672, 224), lmat, pmat,
        yolo_W1, yolo_b1, yolo_W2, yolo_b2)
    top2_flat = top2_3d[:, 0, :2].reshape(2 * bn).astype(jnp.int32)

    o_w1, o_w2, b1g, b2g = _sc_gather(
        idx1_3d.reshape(1024 * bn), idx2_3d.reshape(512 * bn), top2_flat,
        gnn_W1.reshape(64 * 512, 256),
        gnn_W2.reshape(64 * 256, 128),
        gnn_b1, gnn_b2)
    w1g = o_w1.reshape(16, 512, 256)
    w2g = o_w2.reshape(16, 256, 128)

    return _experts(xg, m, w1g, w2g, b1g, b2g,
                    final_W, final_b.reshape(1, 64))


# speedup vs baseline: 7.1634x; 1.1196x over previous
import functools

import jax
import jax.numpy as jnp
import numpy as np
from jax import lax
from jax.experimental import pallas as pl
from jax.experimental.pallas import tpu as pltpu
from jax.experimental.pallas import tpu_sc as plsc

F32 = jnp.float32

_hdot = functools.partial(jnp.dot, precision=lax.Precision.HIGHEST,
                          preferred_element_type=F32)
_fdot = functools.partial(jnp.dot, precision=lax.Precision.DEFAULT,
                          preferred_element_type=F32)


BF16 = jnp.bfloat16


def _split2(v):
    hi = v.astype(BF16)
    lo = (v - hi.astype(F32)).astype(BF16)
    return hi, lo


def _bdot(a, b):
    return jnp.dot(a, b, preferred_element_type=F32)


def _dot3(a1, a2, bhi, blo):
    return (_bdot(a1, bhi) + _bdot(a1, blo)) + _bdot(a2, bhi)


def _route_body(x_ref, lmat_ref, pmat_ref, w1hi_ref, w1lo_ref, b1_ref,
                w2hi_ref, w2lo_ref, b2_ref,
                xg_ref, top2_ref, m_ref, idx1_ref, idx2_ref):
    xb = x_ref[0]
    x1, x2 = _split2(xb)
    lm = lmat_ref[...]
    z = _bdot(lm, x1) + _bdot(lm, x2)
    z1, z2 = _split2(z)
    pm = pmat_ref[...]
    pooled = (_bdot(z1, pm) + _bdot(z2, pm)) * (1.0 / 196.0)

    p1, p2 = _split2(pooled)
    accs = [None] * 4
    for a in range(48):
        d = _dot3(p1[a:a + 1, :], p2[a:a + 1, :], w1hi_ref[a], w1lo_ref[a])
        g = a % 4
        accs[g] = d if accs[g] is None else accs[g] + d
    acc = b1_ref[...] + ((accs[0] + accs[1]) + (accs[2] + accs[3]))
    f = jax.nn.relu(acc)
    f1, f2 = _split2(f)
    lg = _dot3(f1, f2, w2hi_ref[...], w2lo_ref[...]) + b2_ref[...]

    io64 = lax.broadcasted_iota(jnp.int32, (1, 64), 1)
    m1 = jnp.max(lg, axis=1, keepdims=True)
    i1 = jnp.min(jnp.where(lg == m1, io64, 64), axis=1, keepdims=True)
    lg2 = jnp.where(io64 == i1, F32(-1e30), lg)
    m2 = jnp.max(lg2, axis=1, keepdims=True)
    i2 = jnp.min(jnp.where(lg2 == m2, io64, 64), axis=1, keepdims=True)
    io128 = lax.broadcasted_iota(jnp.int32, (1, 128), 1)
    top2_ref[0] = jnp.where(io128 == 0,
                            jnp.broadcast_to(i1, (1, 128)),
                            jnp.broadcast_to(i2, (1, 128)))

    ioa = lax.broadcasted_iota(jnp.int32, (1, 1024), 1)
    e_sel = jnp.where(ioa < 512,
                      jnp.broadcast_to(i1, (1, 1024)),
                      jnp.broadcast_to(i2, (1, 1024)))
    idx1_ref[0] = e_sel * 512 + (ioa & 511)
    iob = lax.broadcasted_iota(jnp.int32, (1, 512), 1)
    e_selb = jnp.where(iob < 256,
                       jnp.broadcast_to(i1, (1, 512)),
                       jnp.broadcast_to(i2, (1, 512)))
    idx2_ref[0] = e_selb * 256 + (iob & 255)

    parts = [f]
    for k in range(4):
        q = f[:, k * 128:(k + 1) * 128]
        parts.append(jnp.pad(q, ((0, 0), (0, 384))))
    xg = jnp.concatenate(parts, axis=0)
    xg_ref[0] = xg

    diff = xg[:, None, :] - xg[None, :, :]
    d2 = jnp.sum(diff * diff, axis=-1)
    io5c = lax.broadcasted_iota(jnp.int32, (5, 5), 1)
    io5r = lax.broadcasted_iota(jnp.int32, (5, 5), 0)
    mx = jnp.max(d2, axis=1, keepdims=True)
    excl = jnp.max(jnp.where(d2 == mx, io5c, -1), axis=1, keepdims=True)
    m_ref[0] = ((io5c != excl).astype(F32) + (io5c == io5r).astype(F32))


def _route(x3, lmat, pmat, w1, b1, w2, b2):
    bn = x3.shape[0]
    w1b = w1.reshape(48, 16, 512)
    w1hi = w1b.astype(BF16)
    w1lo = (w1b - w1hi.astype(F32)).astype(BF16)
    w2hi = w2.astype(BF16)
    w2lo = (w2 - w2hi.astype(F32)).astype(BF16)
    return pl.pallas_call(
        _route_body,
        grid=(bn,),
        in_specs=[
            pl.BlockSpec((1, 672, 224), lambda b: (b, 0, 0)),
            pl.BlockSpec((48, 672), lambda b: (0, 0)),
            pl.BlockSpec((224, 16), lambda b: (0, 0)),
            pl.BlockSpec((48, 16, 512), lambda b: (0, 0, 0)),
            pl.BlockSpec((48, 16, 512), lambda b: (0, 0, 0)),
            pl.BlockSpec((1, 512), lambda b: (0, 0)),
            pl.BlockSpec((512, 64), lambda b: (0, 0)),
            pl.BlockSpec((512, 64), lambda b: (0, 0)),
            pl.BlockSpec((1, 64), lambda b: (0, 0)),
        ],
        out_specs=[
            pl.BlockSpec((1, 5, 512), lambda b: (b, 0, 0)),
            pl.BlockSpec((1, 1, 128), lambda b: (b, 0, 0)),
            pl.BlockSpec((1, 5, 5), lambda b: (b, 0, 0)),
            pl.BlockSpec((1, 1, 1024), lambda b: (b, 0, 0)),
            pl.BlockSpec((1, 1, 512), lambda b: (b, 0, 0)),
        ],
        out_shape=[
            jax.ShapeDtypeStruct((bn, 5, 512), F32),
            jax.ShapeDtypeStruct((bn, 1, 128), jnp.int32),
            jax.ShapeDtypeStruct((bn, 5, 5), F32),
            jax.ShapeDtypeStruct((bn, 1, 1024), jnp.int32),
            jax.ShapeDtypeStruct((bn, 1, 512), jnp.int32),
        ],
    )(x3, lmat.astype(BF16), pmat.astype(BF16), w1hi, w1lo,
      b1.reshape(1, 512), w2hi, w2lo, b2.reshape(1, 64))


def _sc_gather(idx1_flat, idx2_flat, top2_flat, w1_view, w2_view, b1, b2):
    mesh = plsc.VectorSubcoreMesh(core_axis_name="c", subcore_axis_name="s")

    @functools.partial(
        pl.kernel,
        out_type=(
            jax.ShapeDtypeStruct((8192, 256), F32),
            jax.ShapeDtypeStruct((4096, 128), F32),
            jax.ShapeDtypeStruct((16, 256), F32),
            jax.ShapeDtypeStruct((16, 128), F32),
        ),
        mesh=mesh,
        scratch_types=(
            pltpu.VMEM((128,), jnp.int32),
            pltpu.VMEM((128,), jnp.int32),
            pltpu.VMEM((128,), jnp.int32),
            pltpu.VMEM((16,), jnp.int32),
            pltpu.VMEM((256, 256), F32),
            pltpu.VMEM((128, 128), F32),
            pltpu.VMEM((16, 256), F32),
            pltpu.VMEM((16, 128), F32),
            pltpu.SemaphoreType.DMA,
            pltpu.SemaphoreType.DMA,
        ),
    )
    def gather_k(idx1_hbm, idx2_hbm, top2_hbm, w1_hbm, w2_hbm, b1_hbm, b2_hbm,
                 o_w1, o_w2, o_b1, o_b2,
                 ia_v, ib_v, ic_v, e_v, rows1_v, rows2_v, b1_v, b2_v,
                 sem, semb):
        wid = lax.axis_index("s") * 2 + lax.axis_index("c")
        pltpu.sync_copy(idx1_hbm.at[pl.ds(wid * 256, 128)], ia_v)
        pltpu.sync_copy(idx1_hbm.at[pl.ds(wid * 256 + 128, 128)], ib_v)
        pltpu.sync_copy(idx2_hbm.at[pl.ds(wid * 128, 128)], ic_v)
        c1 = pltpu.async_copy(w1_hbm.at[ia_v], rows1_v.at[pl.ds(0, 128)], sem)
        c2 = pltpu.async_copy(w1_hbm.at[ib_v], rows1_v.at[pl.ds(128, 128)], sem)
        c3 = pltpu.async_copy(w2_hbm.at[ic_v], rows2_v, sem)
        c1.wait()
        c2.wait()
        c3.wait()
        pltpu.sync_copy(rows1_v, o_w1.at[pl.ds(wid * 256, 256)])
        pltpu.sync_copy(rows2_v, o_w2.at[pl.ds(wid * 128, 128)])

        @pl.when(wid == 0)
        def _():
            pltpu.sync_copy(top2_hbm, e_v)
            pltpu.async_copy(b1_hbm.at[e_v], b1_v, semb).wait()
            pltpu.sync_copy(b1_v, o_b1)

        @pl.when(wid == 1)
        def _():
            pltpu.sync_copy(top2_hbm, e_v)
            pltpu.async_copy(b2_hbm.at[e_v], b2_v, semb).wait()
            pltpu.sync_copy(b2_v, o_b2)

    return gather_k(idx1_flat, idx2_flat, top2_flat, w1_view, w2_view, b1, b2)


def _experts_body(xg_ref, m_ref, w1_ref, w2_ref, b1_ref, b2_ref,
                  fw_ref, fb_ref, o_ref, acc_ref):
    t = pl.program_id(0)
    xg = xg_ref[0]
    mm = m_ref[0]
    xw = _fdot(xg, w1_ref[0])
    h = jax.nn.relu(_fdot(mm, xw) * 0.2 + b1_ref[pl.ds(t, 1), :])
    h2 = _fdot(mm, _fdot(h, w2_ref[0])) * 0.2 + b2_ref[pl.ds(t, 1), :]
    acc_ref[pl.ds(t, 1), :] = jnp.mean(h2, axis=0, keepdims=True)

    @pl.when(t == 15)
    def _():
        comb = acc_ref[...]
        fin = _fdot(comb, fw_ref[...])
        io5c = lax.broadcasted_iota(jnp.int32, (5, 5), 1)
        io5r = lax.broadcasted_iota(jnp.int32, (5, 5), 0)
        c7 = mm - (io5c == io5r).astype(F32)
        c7p = jnp.pad(c7, ((0, 11), (0, 11)))
        r16 = lax.broadcasted_iota(jnp.int32, (16, 16), 0)
        c16 = lax.broadcasted_iota(jnp.int32, (16, 16), 1)
        diag = jnp.where(r16 == c16,
                         jnp.where(r16 < 5, F32(0.2), F32(1.0)), F32(0.0))
        mf = diag + c7p * 0.2
        fin2 = _fdot(mf, fin) + fb_ref[...]
        r8 = lax.broadcasted_iota(jnp.int32, (8, 16), 0)
        c8 = lax.broadcasted_iota(jnp.int32, (8, 16), 1)
        pairmean = ((c8 == 2 * r8) | (c8 == 2 * r8 + 1)).astype(F32)
        o_ref[...] = _fdot(pairmean, fin2) * 0.5


def _experts(xg, m, w1g, w2g, b1g, b2g, fw, fb):
    return pl.pallas_call(
        _experts_body,
        grid=(16,),
        in_specs=[
            pl.BlockSpec((1, 5, 512), lambda t: (t // 2, 0, 0)),
            pl.BlockSpec((1, 5, 5), lambda t: (t // 2, 0, 0)),
            pl.BlockSpec((1, 512, 256), lambda t: (t, 0, 0)),
            pl.BlockSpec((1, 256, 128), lambda t: (t, 0, 0)),
            pl.BlockSpec((16, 256), lambda t: (0, 0)),
            pl.BlockSpec((16, 128), lambda t: (0, 0)),
            pl.BlockSpec((128, 64), lambda t: (0, 0)),
            pl.BlockSpec((1, 64), lambda t: (0, 0)),
        ],
        out_specs=pl.BlockSpec((8, 64), lambda t: (0, 0)),
        out_shape=jax.ShapeDtypeStruct((8, 64), F32),
        scratch_shapes=[pltpu.VMEM((16, 128), F32)],
    )(xg, m, w1g, w2g, b1g, b2g, fw, fb)


def _make_pool_consts():
    lmat = np.zeros((48, 672), dtype=np.float32)
    for a in range(48):
        ch, i = divmod(a, 16)
        lmat[a, ch * 224 + i * 14:(ch * 224 + (i + 1) * 14)] = 1.0
    pmat = np.zeros((224, 16), dtype=np.float32)
    for rr in range(224):
        pmat[rr, rr // 14] = 1.0
    return jnp.asarray(lmat), jnp.asarray(pmat)


def kernel(x, yolo_W1, yolo_b1, yolo_W2, yolo_b2,
           gnn_W1, gnn_b1, gnn_W2, gnn_b2, final_W, final_b):
    bn = x.shape[0]
    lmat, pmat = _make_pool_consts()
    xg, top2_3d, m, idx1_3d, idx2_3d = _route(
        x.reshape(bn, 672, 224), lmat, pmat,
        yolo_W1, yolo_b1, yolo_W2, yolo_b2)
    top2_flat = top2_3d[:, 0, :2].reshape(2 * bn).astype(jnp.int32)

    o_w1, o_w2, b1g, b2g = _sc_gather(
        idx1_3d.reshape(1024 * bn), idx2_3d.reshape(512 * bn), top2_flat,
        gnn_W1.reshape(64 * 512, 256),
        gnn_W2.reshape(64 * 256, 128),
        gnn_b1, gnn_b2)
    w1g = o_w1.reshape(16, 512, 256)
    w2g = o_w2.reshape(16, 256, 128)

    return _experts(xg, m, w1g, w2g, b1g, b2g,
                    final_W, final_b.reshape(1, 64))


# speedup vs baseline: 7.2368x; 1.0102x over previous
import functools

import jax
import jax.numpy as jnp
import numpy as np
from jax import lax
from jax.experimental import pallas as pl
from jax.experimental.pallas import tpu as pltpu
from jax.experimental.pallas import tpu_sc as plsc

F32 = jnp.float32

_hdot = functools.partial(jnp.dot, precision=lax.Precision.HIGHEST,
                          preferred_element_type=F32)
_fdot = functools.partial(jnp.dot, precision=lax.Precision.DEFAULT,
                          preferred_element_type=F32)


BF16 = jnp.bfloat16


def _split2(v):
    hi = v.astype(BF16)
    lo = (v - hi.astype(F32)).astype(BF16)
    return hi, lo


def _bdot(a, b):
    return jnp.dot(a, b, preferred_element_type=F32)


def _dot3(a1, a2, bhi, blo):
    return (_bdot(a1, bhi) + _bdot(a1, blo)) + _bdot(a2, bhi)


def _route_body(x_ref, lmat_ref, pmat_ref, w1hi_ref, w1lo_ref, b1_ref,
                w2hi_ref, w2lo_ref, b2_ref,
                xg_ref, top2_ref, m_ref, idx1_ref, idx2_ref):
    xb = x_ref[0]
    x1, x2 = _split2(xb)
    lm = lmat_ref[...]
    z = _bdot(lm, x1) + _bdot(lm, x2)
    z1, z2 = _split2(z)
    pm = pmat_ref[...]
    pooled = (_bdot(z1, pm) + _bdot(z2, pm)) * (1.0 / 196.0)

    p1, p2 = _split2(pooled)
    accs = [None] * 4
    for a in range(48):
        d = _dot3(p1[a:a + 1, :], p2[a:a + 1, :], w1hi_ref[a], w1lo_ref[a])
        g = a % 4
        accs[g] = d if accs[g] is None else accs[g] + d
    acc = b1_ref[...] + ((accs[0] + accs[1]) + (accs[2] + accs[3]))
    f = jax.nn.relu(acc)
    f1, f2 = _split2(f)
    lg = _dot3(f1, f2, w2hi_ref[...], w2lo_ref[...]) + b2_ref[...]

    io64 = lax.broadcasted_iota(jnp.int32, (1, 64), 1)
    m1 = jnp.max(lg, axis=1, keepdims=True)
    i1 = jnp.min(jnp.where(lg == m1, io64, 64), axis=1, keepdims=True)
    lg2 = jnp.where(io64 == i1, F32(-1e30), lg)
    m2 = jnp.max(lg2, axis=1, keepdims=True)
    i2 = jnp.min(jnp.where(lg2 == m2, io64, 64), axis=1, keepdims=True)
    io128 = lax.broadcasted_iota(jnp.int32, (1, 128), 1)
    top2_ref[0] = jnp.where(io128 == 0,
                            jnp.broadcast_to(i1, (1, 128)),
                            jnp.broadcast_to(i2, (1, 128)))

    ioa = lax.broadcasted_iota(jnp.int32, (1, 1024), 1)
    e_sel = jnp.where(ioa < 512,
                      jnp.broadcast_to(i1, (1, 1024)),
                      jnp.broadcast_to(i2, (1, 1024)))
    idx1_ref[0] = e_sel * 512 + (ioa & 511)
    iob = lax.broadcasted_iota(jnp.int32, (1, 512), 1)
    e_selb = jnp.where(iob < 256,
                       jnp.broadcast_to(i1, (1, 512)),
                       jnp.broadcast_to(i2, (1, 512)))
    idx2_ref[0] = e_selb * 256 + (iob & 255)

    parts = [f]
    for k in range(4):
        q = f[:, k * 128:(k + 1) * 128]
        parts.append(jnp.pad(q, ((0, 0), (0, 384))))
    xg = jnp.concatenate(parts, axis=0)
    xg_ref[0] = xg

    diff = xg[:, None, :] - xg[None, :, :]
    d2 = jnp.sum(diff * diff, axis=-1)
    io5c = lax.broadcasted_iota(jnp.int32, (5, 5), 1)
    io5r = lax.broadcasted_iota(jnp.int32, (5, 5), 0)
    mx = jnp.max(d2, axis=1, keepdims=True)
    excl = jnp.max(jnp.where(d2 == mx, io5c, -1), axis=1, keepdims=True)
    m_ref[0] = ((io5c != excl).astype(F32) + (io5c == io5r).astype(F32))


def _route(x3, lmat, pmat, w1, b1, w2, b2):
    bn = x3.shape[0]
    w1b = w1.reshape(48, 16, 512)
    w1hi = w1b.astype(BF16)
    w1lo = (w1b - w1hi.astype(F32)).astype(BF16)
    w2hi = w2.astype(BF16)
    w2lo = (w2 - w2hi.astype(F32)).astype(BF16)
    return pl.pallas_call(
        _route_body,
        grid=(bn,),
        in_specs=[
            pl.BlockSpec((1, 672, 224), lambda b: (b, 0, 0)),
            pl.BlockSpec((48, 672), lambda b: (0, 0)),
            pl.BlockSpec((224, 16), lambda b: (0, 0)),
            pl.BlockSpec((48, 16, 512), lambda b: (0, 0, 0)),
            pl.BlockSpec((48, 16, 512), lambda b: (0, 0, 0)),
            pl.BlockSpec((1, 512), lambda b: (0, 0)),
            pl.BlockSpec((512, 64), lambda b: (0, 0)),
            pl.BlockSpec((512, 64), lambda b: (0, 0)),
            pl.BlockSpec((1, 64), lambda b: (0, 0)),
        ],
        out_specs=[
            pl.BlockSpec((1, 5, 512), lambda b: (b, 0, 0)),
            pl.BlockSpec((1, 1, 128), lambda b: (b, 0, 0)),
            pl.BlockSpec((1, 5, 5), lambda b: (b, 0, 0)),
            pl.BlockSpec((1, 1, 1024), lambda b: (b, 0, 0)),
            pl.BlockSpec((1, 1, 512), lambda b: (b, 0, 0)),
        ],
        out_shape=[
            jax.ShapeDtypeStruct((bn, 5, 512), F32),
            jax.ShapeDtypeStruct((bn, 1, 128), jnp.int32),
            jax.ShapeDtypeStruct((bn, 5, 5), F32),
            jax.ShapeDtypeStruct((bn, 1, 1024), jnp.int32),
            jax.ShapeDtypeStruct((bn, 1, 512), jnp.int32),
        ],
    )(x3, lmat.astype(BF16), pmat.astype(BF16), w1hi, w1lo,
      b1.reshape(1, 512), w2hi, w2lo, b2.reshape(1, 64))


def _sc_gather(idx1_flat, idx2_flat, top2_flat, w1_view, w2_view, b1, b2):
    mesh = plsc.VectorSubcoreMesh(core_axis_name="c", subcore_axis_name="s")

    @functools.partial(
        pl.kernel,
        out_type=(
            jax.ShapeDtypeStruct((8192, 256), F32),
            jax.ShapeDtypeStruct((4096, 128), F32),
            jax.ShapeDtypeStruct((16, 256), F32),
            jax.ShapeDtypeStruct((16, 128), F32),
        ),
        mesh=mesh,
        scratch_types=(
            pltpu.VMEM((128,), jnp.int32),
            pltpu.VMEM((128,), jnp.int32),
            pltpu.VMEM((128,), jnp.int32),
            pltpu.VMEM((16,), jnp.int32),
            pltpu.VMEM((256, 256), F32),
            pltpu.VMEM((128, 128), F32),
            pltpu.VMEM((16, 256), F32),
            pltpu.VMEM((16, 128), F32),
            pltpu.SemaphoreType.DMA,
            pltpu.SemaphoreType.DMA,
            pltpu.SemaphoreType.DMA,
        ),
    )
    def gather_k(idx1_hbm, idx2_hbm, top2_hbm, w1_hbm, w2_hbm, b1_hbm, b2_hbm,
                 o_w1, o_w2, o_b1, o_b2,
                 ia_v, ib_v, ic_v, e_v, rows1_v, rows2_v, b1_v, b2_v,
                 sem, semb, semo):
        wid = lax.axis_index("s") * 2 + lax.axis_index("c")
        pltpu.sync_copy(idx1_hbm.at[pl.ds(wid * 256, 128)], ia_v)
        pltpu.sync_copy(idx1_hbm.at[pl.ds(wid * 256 + 128, 128)], ib_v)
        pltpu.sync_copy(idx2_hbm.at[pl.ds(wid * 128, 128)], ic_v)
        c1 = pltpu.async_copy(w1_hbm.at[ia_v], rows1_v.at[pl.ds(0, 128)], sem)
        c2 = pltpu.async_copy(w1_hbm.at[ib_v], rows1_v.at[pl.ds(128, 128)], sem)
        c3 = pltpu.async_copy(w2_hbm.at[ic_v], rows2_v, sem)
        c1.wait()
        o1 = pltpu.async_copy(rows1_v.at[pl.ds(0, 128)],
                              o_w1.at[pl.ds(wid * 256, 128)], semo)
        c2.wait()
        o2 = pltpu.async_copy(rows1_v.at[pl.ds(128, 128)],
                              o_w1.at[pl.ds(wid * 256 + 128, 128)], semo)
        c3.wait()
        o3 = pltpu.async_copy(rows2_v, o_w2.at[pl.ds(wid * 128, 128)], semo)
        o1.wait()
        o2.wait()
        o3.wait()

        @pl.when(wid == 0)
        def _():
            pltpu.sync_copy(top2_hbm, e_v)
            pltpu.async_copy(b1_hbm.at[e_v], b1_v, semb).wait()
            pltpu.sync_copy(b1_v, o_b1)

        @pl.when(wid == 1)
        def _():
            pltpu.sync_copy(top2_hbm, e_v)
            pltpu.async_copy(b2_hbm.at[e_v], b2_v, semb).wait()
            pltpu.sync_copy(b2_v, o_b2)

    return gather_k(idx1_flat, idx2_flat, top2_flat, w1_view, w2_view, b1, b2)


def _experts_body(xg_ref, m_ref, w1_ref, w2_ref, b1_ref, b2_ref,
                  fw_ref, fb_ref, o_ref, acc_ref):
    t = pl.program_id(0)
    xg = xg_ref[0]
    mm = m_ref[0]
    xw = _fdot(xg, w1_ref[0])
    h = jax.nn.relu(_fdot(mm, xw) * 0.2 + b1_ref[pl.ds(t, 1), :])
    h2 = _fdot(mm, _fdot(h, w2_ref[0])) * 0.2 + b2_ref[pl.ds(t, 1), :]
    acc_ref[pl.ds(t, 1), :] = jnp.mean(h2, axis=0, keepdims=True)

    @pl.when(t == 15)
    def _():
        comb = acc_ref[...]
        fin = _fdot(comb, fw_ref[...])
        io5c = lax.broadcasted_iota(jnp.int32, (5, 5), 1)
        io5r = lax.broadcasted_iota(jnp.int32, (5, 5), 0)
        c7 = mm - (io5c == io5r).astype(F32)
        c7p = jnp.pad(c7, ((0, 11), (0, 11)))
        r16 = lax.broadcasted_iota(jnp.int32, (16, 16), 0)
        c16 = lax.broadcasted_iota(jnp.int32, (16, 16), 1)
        diag = jnp.where(r16 == c16,
                         jnp.where(r16 < 5, F32(0.2), F32(1.0)), F32(0.0))
        mf = diag + c7p * 0.2
        fin2 = _fdot(mf, fin) + fb_ref[...]
        r8 = lax.broadcasted_iota(jnp.int32, (8, 16), 0)
        c8 = lax.broadcasted_iota(jnp.int32, (8, 16), 1)
        pairmean = ((c8 == 2 * r8) | (c8 == 2 * r8 + 1)).astype(F32)
        o_ref[...] = _fdot(pairmean, fin2) * 0.5


def _experts(xg, m, w1g, w2g, b1g, b2g, fw, fb):
    return pl.pallas_call(
        _experts_body,
        grid=(16,),
        in_specs=[
            pl.BlockSpec((1, 5, 512), lambda t: (t // 2, 0, 0)),
            pl.BlockSpec((1, 5, 5), lambda t: (t // 2, 0, 0)),
            pl.BlockSpec((1, 512, 256), lambda t: (t, 0, 0)),
            pl.BlockSpec((1, 256, 128), lambda t: (t, 0, 0)),
            pl.BlockSpec((16, 256), lambda t: (0, 0)),
            pl.BlockSpec((16, 128), lambda t: (0, 0)),
            pl.BlockSpec((128, 64), lambda t: (0, 0)),
            pl.BlockSpec((1, 64), lambda t: (0, 0)),
        ],
        out_specs=pl.BlockSpec((8, 64), lambda t: (0, 0)),
        out_shape=jax.ShapeDtypeStruct((8, 64), F32),
        scratch_shapes=[pltpu.VMEM((16, 128), F32)],
    )(xg, m, w1g, w2g, b1g, b2g, fw, fb)


def _make_pool_consts():
    lmat = np.zeros((48, 672), dtype=np.float32)
    for a in range(48):
        ch, i = divmod(a, 16)
        lmat[a, ch * 224 + i * 14:(ch * 224 + (i + 1) * 14)] = 1.0
    pmat = np.zeros((224, 16), dtype=np.float32)
    for rr in range(224):
        pmat[rr, rr // 14] = 1.0
    return jnp.asarray(lmat), jnp.asarray(pmat)


def kernel(x, yolo_W1, yolo_b1, yolo_W2, yolo_b2,
           gnn_W1, gnn_b1, gnn_W2, gnn_b2, final_W, final_b):
    bn = x.shape[0]
    lmat, pmat = _make_pool_consts()
    xg, top2_3d, m, idx1_3d, idx2_3d = _route(
        x.reshape(bn, 672, 224), lmat, pmat,
        yolo_W1, yolo_b1, yolo_W2, yolo_b2)
    top2_flat = top2_3d[:, 0, :2].reshape(2 * bn).astype(jnp.int32)

    o_w1, o_w2, b1g, b2g = _sc_gather(
        idx1_3d.reshape(1024 * bn), idx2_3d.reshape(512 * bn), top2_flat,
        gnn_W1.reshape(64 * 512, 256),
        gnn_W2.reshape(64 * 256, 128),
        gnn_b1, gnn_b2)
    w1g = o_w1.reshape(16, 512, 256)
    w2g = o_w2.reshape(16, 256, 128)

    return _experts(xg, m, w1g, w2g, b1g, b2g,
                    final_W, final_b.reshape(1, 64))
